# baseline (speedup 1.0000x reference)
import functools

import jax
import jax.numpy as jnp
from jax import lax
from jax.experimental import pallas as pl
from jax.experimental.pallas import tpu as pltpu
from jax.experimental.pallas import tpu_sc as plsc

_MESH = plsc.VectorSubcoreMesh(core_axis_name="core", subcore_axis_name="subcore")
_NW = 32
_VG = 7813
_VG_MAIN = _VG // _NW
_VG_TAIL = _VG - _VG_MAIN * _NW
_WPT = 6400 // _NW
_IPT = _WPT * 128


def _iota16():
  return lax.iota(jnp.int32, 16)


def _transpose_tile(inbuf, outbuf):
  it = _iota16()
  rows = [it + (16 * k) for k in range(4)]
  for p in range(64):
    for half in range(2):
      c = jnp.full((16,), 2 * p + half, jnp.int32)
      for k in range(4):
        outbuf[p, pl.ds(64 * half + 16 * k, 16)] = plsc.load_gather(
            inbuf, [rows[k], c])


@jax.jit
def _impl(table_t, idx_t):
  @functools.partial(
      pl.kernel,
      out_type=(jax.ShapeDtypeStruct((500032, 128), jnp.float32),
                jax.ShapeDtypeStruct((819200,), jnp.int32)),
      mesh=_MESH,
      scratch_types=[
          pltpu.VMEM((64, 128), jnp.float32),
          pltpu.VMEM((64, 128), jnp.float32),
          pltpu.VMEM((64, 128), jnp.float32),
          pltpu.VMEM((64, 128), jnp.float32),
          pltpu.VMEM((8, 128), jnp.int32),
          pltpu.SemaphoreType.DMA,
          pltpu.SemaphoreType.DMA,
          pltpu.SemaphoreType.DMA,
          pltpu.SemaphoreType.DMA,
      ],
      compiler_params=pltpu.CompilerParams(use_tc_tiling_on_sc=True, needs_layout_passes=False),
  )
  def call_a(tt_hbm, it_hbm, trm_hbm, idxl_hbm, in0, in1, ou0, ou1, ibuf,
             si0, si1, so0, so1):
    wid = lax.axis_index("subcore") * 2 + lax.axis_index("core")
    inb = (in0, in1)
    oub = (ou0, ou1)
    sin = (si0, si1)
    sou = (so0, so1)

    for t in range(7):
      for i in range(4):
        bg = wid + 32 * i
        nh = 8 if t < 6 else 2
        pltpu.sync_copy(it_hbm.at[pl.ds(8 * t, nh), pl.ds(bg * 128, 128)],
                        ibuf.at[pl.ds(0, nh)])
        for hr in range(nh):
          pltpu.sync_copy(
              ibuf.at[hr],
              idxl_hbm.at[pl.ds((8 * t + hr) * 16384 + bg * 128, 128)])

    n_my = jnp.where(wid < _VG_TAIL, _VG_MAIN + 1, _VG_MAIN)

    def start_in(i, s):
      vg = i * _NW + wid
      pltpu.async_copy(tt_hbm.at[pl.ds(0, 64), pl.ds(vg * 128, 128)],
                       inb[s], sin[s])

    def wait_in(s):
      pltpu.make_async_copy(tt_hbm.at[pl.ds(0, 64), pl.ds(0, 128)],
                            inb[s], sin[s]).wait()

    def start_out(i, s):
      vg = i * _NW + wid
      pltpu.async_copy(oub[s], trm_hbm.at[pl.ds(vg * 64, 64), pl.ds(0, 128)],
                       sou[s])

    def wait_out(s):
      pltpu.make_async_copy(oub[s], trm_hbm.at[pl.ds(0, 64), pl.ds(0, 128)],
                            sou[s]).wait()

    start_in(0, 0)
    start_in(1, 1)

    @pl.loop(0, (_VG_MAIN + 2) // 2)
    def _(i2):
      for s in range(2):
        i = i2 * 2 + s

        @pl.when(i < n_my)
        def _():
          wait_in(s)

          @pl.when(i >= 2)
          def _():
            wait_out(s)

          _transpose_tile(inb[s], oub[s])
          start_out(i, s)

          @pl.when(i + 2 < n_my)
          def _():
            start_in(i + 2, s)

    for s in range(2):
      @pl.when(n_my > s)
      def _():
        wait_out(s)

  trm, idxl = call_a(table_t, idx_t)

  @functools.partial(
      pl.kernel,
      out_type=jax.ShapeDtypeStruct((50, 64, 16384), jnp.float32),
      mesh=_MESH,
      scratch_types=[
          pltpu.VMEM((_IPT,), jnp.int32),
          pltpu.VMEM((_IPT,), jnp.int32),
          pltpu.VMEM((128, 128), jnp.float32),
          pltpu.VMEM((128, 128), jnp.float32),
          pltpu.VMEM((64, 128), jnp.float32),
          pltpu.VMEM((64, 128), jnp.float32),
          pltpu.SemaphoreType.DMA,
          pltpu.SemaphoreType.DMA,
          pltpu.SemaphoreType.DMA,
          pltpu.SemaphoreType.DMA,
          pltpu.SemaphoreType.DMA,
      ],
      compiler_params=pltpu.CompilerParams(use_tc_tiling_on_sc=True, needs_layout_passes=False),
  )
  def call_b(trm_hbm, idxl_hbm, out_hbm, idxr, idxp, ga0, ga1, tr0, tr1,
             sidx, sg0, sg1, st0, st1):
    wid = lax.axis_index("subcore") * 2 + lax.axis_index("core")
    gab = (ga0, ga1)
    trb = (tr0, tr1)
    sg = (sg0, sg1)
    st = (st0, st1)

    pltpu.async_copy(idxl_hbm.at[pl.ds(wid * _IPT, _IPT)], idxr, sidx).wait()

    @pl.loop(0, _IPT // 16)
    def _(q):
      idxp[pl.ds(q * 16, 16)] = jnp.right_shift(idxr[pl.ds(q * 16, 16)], 1)

    def start_gather(j, s):
      pltpu.async_copy(trm_hbm.at[idxp.at[pl.ds(j * 128, 128)]], gab[s],
                       sg[s])

    def wait_gather(s):
      pltpu.make_async_copy(trm_hbm.at[idxp.at[pl.ds(0, 128)]],
                            gab[s], sg[s]).wait()

    def start_store(j, s):
      w = wid * _WPT + j
      h = lax.div(w, 128)
      bg = lax.rem(w, 128)
      pltpu.async_copy(trb[s],
                       out_hbm.at[h, pl.ds(0, 64), pl.ds(bg * 128, 128)],
                       st[s])

    def wait_store(s):
      pltpu.make_async_copy(trb[s],
                            out_hbm.at[0, pl.ds(0, 64), pl.ds(0, 128)],
                            st[s]).wait()

    it = _iota16()
    start_gather(0, 0)
    start_gather(1, 1)

    @pl.loop(0, _WPT // 2)
    def _(j2):
      for s in range(2):
        j = j2 * 2 + s
        wait_gather(s)

        @pl.when(j >= 2)
        def _():
          wait_store(s)

        for k in range(8):
          rows = it + (16 * k)
          hoff = jnp.left_shift(
              jnp.bitwise_and(idxr[pl.ds(j * 128 + 16 * k, 16)], 1), 6)
          for e in range(64):
            trb[s][e, pl.ds(16 * k, 16)] = plsc.load_gather(
                gab[s], [rows, hoff + e])

        start_store(j, s)

        @pl.when(j + 2 < _WPT)
        def _():
          start_gather(j + 2, s)

    for s in range(2):
      wait_store(s)

  return call_b(trm, idxl)


def kernel(table, type_index):
  x = _impl(table.T, type_index.T)
  return x.transpose(2, 0, 1)


# speedup vs baseline: 1.0524x; 1.0524x over previous
import functools

import jax
import jax.numpy as jnp
from jax import lax
from jax.experimental import pallas as pl
from jax.experimental.pallas import tpu as pltpu
from jax.experimental.pallas import tpu_sc as plsc

_MESH = plsc.VectorSubcoreMesh(core_axis_name="core", subcore_axis_name="subcore")
_NW = 32
_U = 3906
_U_MAIN = _U // _NW
_U_TAIL = _U - _U_MAIN * _NW
_CP = pltpu.CompilerParams(use_tc_tiling_on_sc=True, needs_layout_passes=False)


def _iota16():
  return lax.iota(jnp.int32, 16)


def _transpose_block(inbuf, outbuf, rows, np_, pb):
  for dp in range(np_):
    p = pb * 8 + dp
    for half in range(2):
      c = jnp.full((16,), 0, jnp.int32) + (2 * p + half)
      for k in range(4):
        outbuf[p, pl.ds(64 * half + 16 * k, 16)] = plsc.load_gather(
            inbuf, [rows[k], c])


@jax.jit
def _impl(table_t, tail2, idx_t):
  @functools.partial(
      pl.kernel,
      out_type=jax.ShapeDtypeStruct((500032, 128), jnp.float32),
      mesh=_MESH,
      scratch_types=[
          pltpu.VMEM((64, 256), jnp.float32),
          pltpu.VMEM((64, 256), jnp.float32),
          pltpu.VMEM((128, 128), jnp.float32),
          pltpu.VMEM((128, 128), jnp.float32),
          pltpu.SemaphoreType.DMA,
          pltpu.SemaphoreType.DMA,
          pltpu.SemaphoreType.DMA,
          pltpu.SemaphoreType.DMA,
      ],
      compiler_params=_CP,
  )
  def call_a(tt_hbm, tail_hbm, trm_hbm, in0, in1, ou0, ou1, si0, si1, so0, so1):
    wid = lax.axis_index("subcore") * 2 + lax.axis_index("core")
    inb = (in0, in1)
    oub = (ou0, ou1)
    sin = (si0, si1)
    sou = (so0, so1)

    n_my = jnp.where(wid < _U_TAIL, _U_MAIN + 1, _U_MAIN)

    def start_in(i, s):
      u = i * _NW + wid
      pltpu.async_copy(tt_hbm.at[pl.ds(0, 64), pl.ds(u * 256, 256)],
                       inb[s], sin[s])

    def wait_in(s):
      pltpu.make_async_copy(tt_hbm.at[pl.ds(0, 64), pl.ds(0, 256)],
                            inb[s], sin[s]).wait()

    def start_out(i, s):
      u = i * _NW + wid
      pltpu.async_copy(oub[s], trm_hbm.at[pl.ds(u * 128, 128), pl.ds(0, 128)],
                       sou[s])

    def wait_out(s):
      pltpu.make_async_copy(oub[s], trm_hbm.at[pl.ds(0, 128), pl.ds(0, 128)],
                            sou[s]).wait()

    it = _iota16()
    rows = [it + (16 * k) for k in range(4)]

    start_in(0, 0)
    start_in(1, 1)

    @pl.loop(0, (_U_MAIN + 2) // 2)
    def _(i2):
      for s in range(2):
        i = i2 * 2 + s

        @pl.when(i < n_my)
        def _():
          wait_in(s)

          @pl.when(i >= 2)
          def _():
            wait_out(s)

          @pl.loop(0, 16)
          def _(pb):
            _transpose_block(inb[s], oub[s], rows, 8, pb)

          start_out(i, s)

          @pl.when(i + 2 < n_my)
          def _():
            start_in(i + 2, s)

    for s in range(2):
      @pl.when(n_my > s)
      def _():
        wait_out(s)

    @pl.when(wid == _NW - 1)
    def _():
      tv = in0.at[pl.ds(0, 32), pl.ds(0, 128)]
      pltpu.sync_copy(tail_hbm, tv)
      pltpu.sync_copy(tv, trm_hbm.at[pl.ds(499968, 32), pl.ds(0, 128)])

  trm = call_a(table_t, tail2)

  @functools.partial(
      pl.kernel,
      out_type=jax.ShapeDtypeStruct((50, 64, 16384), jnp.float32),
      mesh=_MESH,
      scratch_types=[
          pltpu.VMEM((256, 128), jnp.float32),
          pltpu.VMEM((256, 128), jnp.float32),
          pltpu.VMEM((64, 256), jnp.float32),
          pltpu.VMEM((64, 256), jnp.float32),
          pltpu.VMEM((1, 256), jnp.int32),
          pltpu.VMEM((1, 256), jnp.int32),
          pltpu.VMEM((256,), jnp.int32),
          pltpu.VMEM((256,), jnp.int32),
          pltpu.VMEM((256,), jnp.int32),
          pltpu.VMEM((256,), jnp.int32),
          pltpu.SemaphoreType.DMA,
          pltpu.SemaphoreType.DMA,
          pltpu.SemaphoreType.DMA,
          pltpu.SemaphoreType.DMA,
          pltpu.SemaphoreType.DMA,
          pltpu.SemaphoreType.DMA,
      ],
      compiler_params=_CP,
  )
  def call_b(trm_hbm, it_hbm, out_hbm, ga0, ga1, tr0, tr1,
             iw0, iw1, ip0, ip1, ho0, ho1,
             sg0, sg1, st0, st1, sw0, sw1):
    wid = lax.axis_index("subcore") * 2 + lax.axis_index("core")
    gab = (ga0, ga1)
    trb = (tr0, tr1)
    iwb = (iw0, iw1)
    ipb = (ip0, ip1)
    hob = (ho0, ho1)
    sg = (sg0, sg1)
    st = (st0, st1)
    sw = (sw0, sw1)

    def start_idx(h, s):
      bg2 = 2 * wid + s
      pltpu.async_copy(it_hbm.at[pl.ds(h, 1), pl.ds(bg2 * 256, 256)],
                       iwb[s], sw[s])

    def wait_idx(s):
      pltpu.make_async_copy(it_hbm.at[pl.ds(0, 1), pl.ds(0, 256)],
                            iwb[s], sw[s]).wait()

    def prep_idx(s):
      for q in range(16):
        v = iwb[s][0, pl.ds(16 * q, 16)]
        ipb[s][pl.ds(16 * q, 16)] = jnp.right_shift(v, 1)
        hob[s][pl.ds(16 * q, 16)] = jnp.left_shift(jnp.bitwise_and(v, 1), 6)

    def start_gather(s):
      pltpu.async_copy(trm_hbm.at[ipb[s]], gab[s], sg[s])

    def wait_gather(s):
      pltpu.make_async_copy(trm_hbm.at[ipb[s]], gab[s], sg[s]).wait()

    def start_store(h, s):
      bg2 = 2 * wid + s
      pltpu.async_copy(trb[s],
                       out_hbm.at[h, pl.ds(0, 64), pl.ds(bg2 * 256, 256)],
                       st[s])

    def wait_store(s):
      pltpu.make_async_copy(trb[s],
                            out_hbm.at[0, pl.ds(0, 64), pl.ds(0, 256)],
                            st[s]).wait()

    it16 = _iota16()

    for s in range(2):
      start_idx(0, s)
      wait_idx(s)
      prep_idx(s)
      start_gather(s)
      start_idx(1, s)

    @pl.loop(0, 50)
    def _(h):
      for s in range(2):
        wait_gather(s)

        @pl.when(h >= 1)
        def _():
          wait_store(s)

        @pl.loop(0, 16)
        def _(cb):
          rows = it16 + cb * 16
          hoff = hob[s][pl.ds(cb * 16, 16)]
          for e in range(64):
            trb[s][e, pl.ds(cb * 16, 16)] = plsc.load_gather(
                gab[s], [rows, hoff + e])

        start_store(h, s)

        @pl.when(h + 1 < 50)
        def _():
          wait_idx(s)
          prep_idx(s)
          start_gather(s)

          @pl.when(h + 2 < 50)
          def _():
            start_idx(h + 2, s)

    for s in range(2):
      wait_store(s)

  return call_b(trm, idx_t)


def kernel(table, type_index):
  tail2 = table[999936:1000000].reshape(32, 128)
  x = _impl(table.T, tail2, type_index.T)
  return x.transpose(2, 0, 1)
